# baseline (speedup 1.0000x reference)
import jax
import jax.numpy as jnp
from jax import lax
from jax.experimental import pallas as pl
from jax.experimental.pallas import tpu as pltpu
from jax.experimental.pallas import tpu_sc as plsc

NUM_HEADS = 8
N_GLOBAL = 3
SEQ_LEN = 67
NC, NS, LANES = 2, 16, 16
NW = NC * NS
NR = 16
NR_LAST = SEQ_LEN - 3 * NR
VECS = (NR_LAST * SEQ_LEN + LANES - 1) // LANES

OFF_DB = 576
OFF_AB = 584
OFF_GB = 592
TAB_LEN = 2240


def _sc_body(tab_h, out_h, tab_v, chunk_v):
    wid = lax.axis_index("s") * NC + lax.axis_index("c")
    pltpu.sync_copy(tab_h, tab_v)

    h = wid >> 2
    q = wid & 3
    r0 = q * NR
    nrows = jnp.where(q < 3, NR, NR_LAST)
    start = h * 4489 + r0 * 67
    last = start + nrows * 67 - 1
    p0 = start + lax.iota(jnp.int32, LANES)
    zeros = jnp.zeros((LANES,), jnp.float32)
    zi = jnp.zeros((LANES,), jnp.int32)
    db_vec = plsc.load_gather(tab_v, [zi + (OFF_DB + h)])
    ab_vec = plsc.load_gather(tab_v, [zi + (OFF_AB + h)])
    gofs = OFF_GB + h * 201

    @plsc.parallel_loop(0, VECS)
    def _(v):
        p = jnp.minimum(p0 + (v << 4), last)
        rem = p - h * 4489
        i = (rem * 3913) >> 18
        j = rem - i * 67

        si = i - N_GLOBAL
        sj = j - N_GLOBAL
        ri = si >> 3
        fi = si & 7
        rj = sj >> 3
        fj = sj & 7
        dr = jnp.abs(ri - rj)
        df = jnp.abs(fi - fj)
        v_sq = plsc.load_gather(tab_v, [(dr << 6) + (df << 3) + h])
        v_sq = v_sq + jnp.where(ri - fi == rj - fj, db_vec, zeros)
        v_sq = v_sq + jnp.where(ri + fi == rj + fj, ab_vec, zeros)

        is_top = i < N_GLOBAL
        gmid = jnp.where(is_top, i, jnp.minimum(j, N_GLOBAL - 1))
        glast = jnp.where(is_top, j, i)
        v_glob = plsc.load_gather(tab_v, [gmid * 67 + (gofs + glast)])

        in_sq = (i >= N_GLOBAL) & (j >= N_GLOBAL)
        plsc.store_scatter(chunk_v, [i - r0, j],
                           jnp.where(in_sq, v_sq, v_glob))

    @pl.when(q < 3)
    def _():
        pltpu.sync_copy(chunk_v.at[pl.ds(0, NR), :],
                        out_h.at[h, pl.ds(r0, NR)])

    @pl.when(q == 3)
    def _():
        pltpu.sync_copy(chunk_v, out_h.at[h, pl.ds(3 * NR, NR_LAST)])


def kernel(rank_embed, file_embed, diag_bias, antidiag_bias, knight_bias,
           global_bias, rank_diff, file_diff, same_diag, same_antidiag,
           knight_reach):
    drr = jnp.arange(9)[:, None]
    dff = jnp.arange(8)[None, :]
    kmask = (drr * dff == 2).astype(rank_embed.dtype)
    re9 = jnp.concatenate([rank_embed,
                           jnp.zeros((1, NUM_HEADS), rank_embed.dtype)])
    comb = (re9[:, None, :] + file_embed[None, :, :]
            + kmask[:, :, None] * knight_bias)
    tab = jnp.concatenate([
        comb.reshape(-1),
        diag_bias, antidiag_bias,
        global_bias.reshape(-1),
        jnp.zeros((TAB_LEN - OFF_GB - NUM_HEADS * N_GLOBAL * SEQ_LEN,),
                  rank_embed.dtype),
    ])
    out = pl.kernel(
        _sc_body,
        out_type=jax.ShapeDtypeStruct((NUM_HEADS, SEQ_LEN, SEQ_LEN),
                                      jnp.float32),
        mesh=plsc.VectorSubcoreMesh(core_axis_name="c", subcore_axis_name="s",
                                    num_cores=NC, num_subcores=NS),
        compiler_params=pltpu.CompilerParams(needs_layout_passes=False),
        scratch_types=[
            pltpu.VMEM((TAB_LEN,), jnp.float32),
            pltpu.VMEM((NR_LAST, SEQ_LEN), jnp.float32),
        ],
    )(tab)
    return out[None]


# speedup vs baseline: 1.0196x; 1.0196x over previous
import jax
import jax.numpy as jnp
from jax import lax
from jax.experimental import pallas as pl
from jax.experimental.pallas import tpu as pltpu
from jax.experimental.pallas import tpu_sc as plsc

NUM_HEADS = 8
N_GLOBAL = 3
SEQ_LEN = 67
TOTAL = NUM_HEADS * SEQ_LEN * SEQ_LEN
NC, NS, LANES = 2, 16, 16
NW = NC * NS
NR = 16
NR_LAST = SEQ_LEN - 3 * NR
VECS = (NR_LAST * SEQ_LEN + LANES - 1) // LANES

OFF_FE = 72
OFF_DB = 136
OFF_AB = 144
OFF_KB = 152
OFF_GB = 160
TAB_LEN = 1792


def _sc_body(tab_h, out_h, tab_v, chunk_v):
    wid = lax.axis_index("s") * NC + lax.axis_index("c")
    pltpu.sync_copy(tab_h, tab_v)

    h = wid >> 2
    q = wid & 3
    r0 = q * NR
    nrows = jnp.where(q < 3, NR, NR_LAST)
    start = h * 4489 + r0 * 67
    last = start + nrows * 67 - 1
    p0 = start + lax.iota(jnp.int32, LANES)
    zeros = jnp.zeros((LANES,), jnp.float32)
    zi = jnp.zeros((LANES,), jnp.int32)
    idx_db = zi + (OFF_DB + h)
    idx_ab = zi + (OFF_AB + h)
    idx_kb = zi + (OFF_KB + h)

    @plsc.parallel_loop(0, VECS, unroll=2)
    def _(v):
        p = jnp.minimum(p0 + (v << 4), last)
        rem = p - h * 4489
        i = (rem * 3913) >> 18
        j = rem - i * 67

        si = i - N_GLOBAL
        sj = j - N_GLOBAL
        ri = si >> 3
        fi = si & 7
        rj = sj >> 3
        fj = sj & 7
        dr = jnp.abs(ri - rj)
        df = jnp.abs(fi - fj)
        v_sq = (plsc.load_gather(tab_v, [(dr << 3) + h])
                + plsc.load_gather(tab_v, [(df << 3) + (OFF_FE + h)]))
        v_sq = v_sq + jnp.where(ri - fi == rj - fj,
                                plsc.load_gather(tab_v, [idx_db]), zeros)
        v_sq = v_sq + jnp.where(ri + fi == rj + fj,
                                plsc.load_gather(tab_v, [idx_ab]), zeros)
        v_sq = v_sq + jnp.where(dr * df == 2,
                                plsc.load_gather(tab_v, [idx_kb]), zeros)

        is_top = i < N_GLOBAL
        gmid = jnp.where(is_top, i, jnp.minimum(j, N_GLOBAL - 1))
        glast = jnp.where(is_top, j, i)
        v_glob = plsc.load_gather(
            tab_v, [h * 201 + gmid * 67 + (OFF_GB + glast)])

        in_sq = (i >= N_GLOBAL) & (j >= N_GLOBAL)
        plsc.store_scatter(chunk_v, [i - r0, j],
                           jnp.where(in_sq, v_sq, v_glob))

    @pl.when(q < 3)
    def _():
        pltpu.sync_copy(chunk_v.at[pl.ds(0, NR), :],
                        out_h.at[h, pl.ds(r0, NR)])

    @pl.when(q == 3)
    def _():
        pltpu.sync_copy(chunk_v, out_h.at[h, pl.ds(3 * NR, NR_LAST)])


def kernel(rank_embed, file_embed, diag_bias, antidiag_bias, knight_bias,
           global_bias, rank_diff, file_diff, same_diag, same_antidiag,
           knight_reach):
    z8 = jnp.zeros((8,), rank_embed.dtype)
    tab = jnp.concatenate([
        rank_embed.reshape(-1), z8,
        file_embed.reshape(-1),
        diag_bias, antidiag_bias, knight_bias,
        global_bias.reshape(-1),
        jnp.zeros((TAB_LEN - OFF_GB - NUM_HEADS * N_GLOBAL * SEQ_LEN,),
                  rank_embed.dtype),
    ])
    out = pl.kernel(
        _sc_body,
        out_type=jax.ShapeDtypeStruct((NUM_HEADS, SEQ_LEN, SEQ_LEN),
                                      jnp.float32),
        mesh=plsc.VectorSubcoreMesh(core_axis_name="c", subcore_axis_name="s",
                                    num_cores=NC, num_subcores=NS),
        compiler_params=pltpu.CompilerParams(needs_layout_passes=False),
        scratch_types=[
            pltpu.VMEM((TAB_LEN,), jnp.float32),
            pltpu.VMEM((NR_LAST, SEQ_LEN), jnp.float32),
        ],
    )(tab)
    return out[None]
